# baseline (speedup 1.0000x reference)
import functools

import jax
import jax.numpy as jnp
from jax import lax
from jax.experimental import pallas as pl
from jax.experimental.pallas import tpu as pltpu
from jax.experimental.pallas import tpu_sc as plsc

EMB = 32
WIN = 5
NEG = 20
B = 16384
ENT = 2 * WIN + 1 + NEG
NW = 32
GSZ = 128
NCH = ENT * (B // GSZ)
NG = NCH // NW
NB = 8
B4 = B // 4
NE = NEG + 1


def _tr_body(i_ref, o_ref):
    v = i_ref[...]
    o_ref[...] = v.T.reshape(ENT, 8, GSZ)


def _transpose_idx(idxcat):
    return pl.pallas_call(
        _tr_body,
        grid=(B // 1024,),
        in_specs=[pl.BlockSpec((1024, ENT), lambda j: (j, 0))],
        out_specs=pl.BlockSpec((ENT, 8, GSZ), lambda j: (0, j, 0)),
        out_shape=jax.ShapeDtypeStruct((ENT, B // GSZ, GSZ), jnp.int32),
    )(idxcat)


VOCAB = 1000000
NBJ = VOCAB // GSZ + 1
KPW = (NBJ + NW - 1) // NW


@functools.lru_cache(maxsize=1)
def _make_sc_repack():
    mesh = plsc.VectorSubcoreMesh(core_axis_name="c", subcore_axis_name="s")

    @functools.partial(
        pl.kernel,
        mesh=mesh,
        out_type=jax.ShapeDtypeStruct((VOCAB * EMB // GSZ, GSZ), jnp.float32),
        scratch_types=[
            pltpu.VMEM((2, 4, 8, GSZ), jnp.float32),
            pltpu.VMEM((2, 4, 8, GSZ), jnp.float32),
            pltpu.SemaphoreType.DMA((2,)),
            pltpu.SemaphoreType.DMA((2,)),
        ],
        compiler_params=pltpu.CompilerParams(needs_layout_passes=False),
    )
    def _sc_repack(tT3, out, tin, tout, isem, osem):
        wid = lax.axis_index("s") * 2 + lax.axis_index("c")
        iota = lax.iota(jnp.int32, 16)
        idxs = []
        for b in range(8):
            d = iota + 16 * (b & 1)
            idxs.append((d >> 3, d & 7))

        def fire_in(j, s):
            pltpu.async_copy(
                tT3.at[:, :, pl.ds(j * GSZ, GSZ)], tin.at[s], isem.at[s])

        def wait_in(j, s):
            pltpu.make_async_copy(
                tT3.at[:, :, pl.ds(j * GSZ, GSZ)], tin.at[s],
                isem.at[s]).wait()

        def fire_out(j, s, ntiles):
            for i in range(ntiles):
                pltpu.async_copy(
                    tout.at[s, i], out.at[pl.ds(j * EMB + 8 * i, 8)],
                    osem.at[s])

        def wait_out(j, s, ntiles):
            for i in range(ntiles):
                pltpu.make_async_copy(
                    tout.at[s, i], out.at[pl.ds(j * EMB + 8 * i, 8)],
                    osem.at[s]).wait()

        def transpose_block(s):
            for rr in range(EMB):
                for b in range(8):
                    iv, sv = idxs[b]
                    colv = jnp.full((16,), rr * 4 + (b >> 1), jnp.int32)
                    vals = plsc.load_gather(tin.at[s], [iv, sv, colv])
                    tout[s, rr >> 3, rr & 7, pl.ds(16 * b, 16)] = vals

        fire_in(wid, 0)

        def body(k, carry):
            s = lax.rem(k, 2)
            j = wid + NW * k

            @pl.when(k + 1 < KPW - 1)
            def _pf():
                fire_in(j + NW, 1 - s)

            wait_in(j, s)

            @pl.when(k >= 2)
            def _wo():
                wait_out(j - 2 * NW, s, 4)

            transpose_block(s)
            fire_out(j, s, 4)
            return carry

        lax.fori_loop(0, KPW - 1, body, 0)

        jt = wid + NW * (KPW - 1)
        st = lax.rem(KPW - 1, 2)

        @pl.when(jt < NBJ)
        def _tail():
            fire_in(jt, st)
            wait_in(jt, st)
            wait_out(jt - 2 * NW, st, 4)
            transpose_block(st)

            @pl.when(jt < NBJ - 1)
            def _full():
                fire_out(jt, st, 4)
                wait_out(jt, st, 4)

            @pl.when(jt == NBJ - 1)
            def _partial():
                fire_out(jt, st, 2)
                wait_out(jt, st, 2)

            wait_out(jt - NW, 1 - st, 4)

        @pl.when(jt >= NBJ)
        def _drain_no_tail():
            wait_out(jt - 2 * NW, st, 4)
            wait_out(jt - NW, 1 - st, 4)

    return _sc_repack


def _repack_table(tT3):
    return _make_sc_repack()(tT3)


@functools.lru_cache(maxsize=1)
def _make_sc_gather():
    mesh = plsc.VectorSubcoreMesh(core_axis_name="c", subcore_axis_name="s")

    @functools.partial(
        pl.kernel,
        mesh=mesh,
        out_type=jax.ShapeDtypeStruct((ENT, B, EMB), jnp.float32),
        scratch_types=[
            pltpu.VMEM((NB, GSZ), jnp.int32),
            pltpu.VMEM((NB, GSZ, EMB), jnp.float32),
            pltpu.SemaphoreType.DMA((NB,)),
            pltpu.SemaphoreType.DMA((NB,)),
        ],
        compiler_params=pltpu.CompilerParams(use_tc_tiling_on_sc=False),
    )
    def _sc_gather(table, idx3, out, idx_v, rows_v, isem, gsem):
        wid = lax.axis_index("s") * 2 + lax.axis_index("c")
        c0 = wid * NG

        def idx_src(c):
            return idx3.at[c // GSZ, lax.rem(c, GSZ)]

        def fire_idx(c, b):
            pltpu.async_copy(idx_src(c), idx_v.at[b], isem.at[b])

        def wait_idx(c, b):
            pltpu.make_async_copy(idx_src(c), idx_v.at[b], isem.at[b]).wait()

        def fire_gather(b):
            pltpu.async_copy(table.at[idx_v.at[b]], rows_v.at[b], gsem.at[b])

        def wait_gather(b):
            pltpu.make_async_copy(
                table.at[idx_v.at[b]], rows_v.at[b], gsem.at[b]
            ).wait()

        def store(c, b):
            pltpu.sync_copy(
                rows_v.at[b],
                out.at[c // GSZ, pl.ds(lax.rem(c, GSZ) * GSZ, GSZ)],
            )

        for b in range(NB):
            fire_idx(c0 + b, b)
        for b in range(NB):
            wait_idx(c0 + b, b)
            fire_gather(b)

        def outer(o, carry):
            for b in range(NB):
                r = o * NB + b
                c = c0 + r
                wait_gather(b)
                store(c, b)

                @pl.when(r + NB < NG)
                def _refill():
                    fire_idx(c + NB, b)
                    wait_idx(c + NB, b)
                    fire_gather(b)

            return carry

        lax.fori_loop(0, NG // NB, outer, 0)

        rem = NG - (NG // NB) * NB
        for b in range(rem):
            c = c0 + (NG // NB) * NB + b
            wait_gather(b)
            store(c, b)

    return _sc_gather


BQ = 256


def _tc_body(g_ref, w1_ref, b1_ref, w2_ref, b2_ref, o_ref):
    g = g_ref[...].reshape(ENT * BQ, 128)
    h = jax.nn.sigmoid(
        jnp.dot(g, w1_ref[...], preferred_element_type=jnp.float32)
        + b1_ref[...]
    )
    h = h.reshape(ENT, BQ, 128)
    means = h[0]
    for i in range(1, 2 * WIN):
        means = means + h[i]
    means = means * (1.0 / (2 * WIN))
    acc = jnp.dot(means * h[2 * WIN], w2_ref[0],
                  preferred_element_type=jnp.float32)
    for i in range(1, NE):
        acc = acc + jnp.dot(means * h[2 * WIN + i], w2_ref[i],
                            preferred_element_type=jnp.float32)
    logits = acc + b2_ref[...]
    for k in range(4):
        seg = logits[:, NE * k:NE * (k + 1)]
        m = jnp.max(seg, axis=-1, keepdims=True)
        ex = jnp.exp(seg - m)
        o_ref[:, NE * k:NE * (k + 1)] = ex / jnp.sum(ex, axis=-1,
                                                     keepdims=True)


def _tc_dense(g4, W1bd, b1t, W2p, b2t):
    return pl.pallas_call(
        _tc_body,
        grid=(B4 // BQ,),
        in_specs=[
            pl.BlockSpec((ENT, BQ, 128), lambda i: (0, i, 0)),
            pl.BlockSpec((128, 128), lambda i: (0, 0)),
            pl.BlockSpec((1, 128), lambda i: (0, 0)),
            pl.BlockSpec((NE, 128, 4 * NE), lambda i: (0, 0, 0)),
            pl.BlockSpec((1, 4 * NE), lambda i: (0, 0)),
        ],
        out_specs=pl.BlockSpec((BQ, 4 * NE), lambda i: (i, 0)),
        out_shape=jax.ShapeDtypeStruct((B4, 4 * NE), jnp.float32),
    )(g4, W1bd, b1t, W2p, b2t)


def kernel(inputs, target, negatives, emb_table, W1, b1, W2, b2):
    idxcat = jnp.concatenate([inputs, target, negatives], axis=1)
    idxcat = idxcat.astype(jnp.int32)
    idx3 = _transpose_idx(idxcat)
    table_flat = _repack_table(emb_table.T.reshape(4, 8, VOCAB)).reshape(
        VOCAB, EMB)
    gathered = _make_sc_gather()(table_flat, idx3)
    g4 = gathered.reshape(ENT, B4, GSZ)

    eye4 = jnp.eye(4, dtype=jnp.float32)
    W1bd = jnp.kron(eye4, W1)
    b1t = jnp.tile(b1, 4).reshape(1, 128)
    W2p = jnp.stack(
        [jnp.kron(eye4, W2[i * EMB:(i + 1) * EMB, :]) for i in range(NE)]
    )
    b2t = jnp.tile(b2, 4).reshape(1, 4 * NE)

    out84 = _tc_dense(g4, W1bd, b1t, W2p, b2t)
    return out84.reshape(B, NE)


# speedup vs baseline: 2.3084x; 2.3084x over previous
import functools

import jax
import jax.numpy as jnp
from jax import lax
from jax.experimental import pallas as pl
from jax.experimental.pallas import tpu as pltpu
from jax.experimental.pallas import tpu_sc as plsc

EMB = 32
WIN = 5
NEG = 20
B = 16384
ENT = 2 * WIN + 1 + NEG
NW = 32
GSZ = 128
NCH = ENT * (B // GSZ)
NG = NCH // NW
NB = 8
B4 = B // 4
NE = NEG + 1


def _tr_body(i_ref, o_ref):
    v = i_ref[...]
    g = (v & -8192) + ((v & 2047) << 2) + ((v & 8191) >> 11)
    o_ref[...] = g.T.reshape(ENT, 8, GSZ)


def _transpose_idx(idxcat):
    return pl.pallas_call(
        _tr_body,
        grid=(B // 1024,),
        in_specs=[pl.BlockSpec((1024, ENT), lambda j: (j, 0))],
        out_specs=pl.BlockSpec((ENT, 8, GSZ), lambda j: (0, j, 0)),
        out_shape=jax.ShapeDtypeStruct((ENT, B // GSZ, GSZ), jnp.int32),
    )(idxcat)


VOCAB = 1000000
RB = 8192
NRB = (VOCAB + RB - 1) // RB
TROWS = NRB * RB


def _rp_body(t_ref, o_ref):
    q = RB // 4
    for a in range(4):
        o_ref[:, a * EMB:(a + 1) * EMB] = t_ref[:, a * q:(a + 1) * q].T


def _repack_table(tT):
    return pl.pallas_call(
        _rp_body,
        grid=(NRB,),
        in_specs=[pl.BlockSpec((EMB, RB), lambda j: (0, j))],
        out_specs=pl.BlockSpec((RB // 4, 128), lambda j: (j, 0)),
        out_shape=jax.ShapeDtypeStruct((TROWS // 4, 128), jnp.float32),
    )(tT)


@functools.lru_cache(maxsize=1)
def _make_sc_gather():
    mesh = plsc.VectorSubcoreMesh(core_axis_name="c", subcore_axis_name="s")

    @functools.partial(
        pl.kernel,
        mesh=mesh,
        out_type=jax.ShapeDtypeStruct((ENT, B, EMB), jnp.float32),
        scratch_types=[
            pltpu.VMEM((NB, GSZ), jnp.int32),
            pltpu.VMEM((NB, GSZ, EMB), jnp.float32),
            pltpu.SemaphoreType.DMA((NB,)),
            pltpu.SemaphoreType.DMA((NB,)),
        ],
        compiler_params=pltpu.CompilerParams(use_tc_tiling_on_sc=False),
    )
    def _sc_gather(table, idx3, out, idx_v, rows_v, isem, gsem):
        wid = lax.axis_index("s") * 2 + lax.axis_index("c")
        c0 = wid * NG

        def idx_src(c):
            return idx3.at[c // GSZ, lax.rem(c, GSZ)]

        def fire_idx(c, b):
            pltpu.async_copy(idx_src(c), idx_v.at[b], isem.at[b])

        def wait_idx(c, b):
            pltpu.make_async_copy(idx_src(c), idx_v.at[b], isem.at[b]).wait()

        def fire_gather(b):
            pltpu.async_copy(table.at[idx_v.at[b]], rows_v.at[b], gsem.at[b])

        def wait_gather(b):
            pltpu.make_async_copy(
                table.at[idx_v.at[b]], rows_v.at[b], gsem.at[b]
            ).wait()

        def store(c, b):
            pltpu.sync_copy(
                rows_v.at[b],
                out.at[c // GSZ, pl.ds(lax.rem(c, GSZ) * GSZ, GSZ)],
            )

        for b in range(NB):
            fire_idx(c0 + b, b)
        for b in range(NB):
            wait_idx(c0 + b, b)
            fire_gather(b)

        def outer(o, carry):
            for b in range(NB):
                r = o * NB + b
                c = c0 + r
                wait_gather(b)
                store(c, b)

                @pl.when(r + NB < NG)
                def _refill():
                    fire_idx(c + NB, b)
                    wait_idx(c + NB, b)
                    fire_gather(b)

            return carry

        lax.fori_loop(0, NG // NB, outer, 0)

        rem = NG - (NG // NB) * NB
        for b in range(rem):
            c = c0 + (NG // NB) * NB + b
            wait_gather(b)
            store(c, b)

    return _sc_gather


BQ = 256


def _tc_body(g_ref, w1_ref, b1_ref, w2_ref, b2_ref, o_ref):
    g = g_ref[...].reshape(ENT * BQ, 128)
    h = jax.nn.sigmoid(
        jnp.dot(g, w1_ref[...], preferred_element_type=jnp.float32)
        + b1_ref[...]
    )
    h = h.reshape(ENT, BQ, 128)
    means = h[0]
    for i in range(1, 2 * WIN):
        means = means + h[i]
    means = means * (1.0 / (2 * WIN))
    acc = jnp.dot(means * h[2 * WIN], w2_ref[0],
                  preferred_element_type=jnp.float32)
    for i in range(1, NE):
        acc = acc + jnp.dot(means * h[2 * WIN + i], w2_ref[i],
                            preferred_element_type=jnp.float32)
    logits = acc + b2_ref[...]
    for k in range(4):
        seg = logits[:, NE * k:NE * (k + 1)]
        m = jnp.max(seg, axis=-1, keepdims=True)
        ex = jnp.exp(seg - m)
        o_ref[:, NE * k:NE * (k + 1)] = ex / jnp.sum(ex, axis=-1,
                                                     keepdims=True)


def _tc_dense(g4, W1bd, b1t, W2p, b2t):
    return pl.pallas_call(
        _tc_body,
        grid=(B4 // BQ,),
        in_specs=[
            pl.BlockSpec((ENT, BQ, 128), lambda i: (0, i, 0)),
            pl.BlockSpec((128, 128), lambda i: (0, 0)),
            pl.BlockSpec((1, 128), lambda i: (0, 0)),
            pl.BlockSpec((NE, 128, 4 * NE), lambda i: (0, 0, 0)),
            pl.BlockSpec((1, 4 * NE), lambda i: (0, 0)),
        ],
        out_specs=pl.BlockSpec((BQ, 4 * NE), lambda i: (i, 0)),
        out_shape=jax.ShapeDtypeStruct((B4, 4 * NE), jnp.float32),
    )(g4, W1bd, b1t, W2p, b2t)


def kernel(inputs, target, negatives, emb_table, W1, b1, W2, b2):
    idxcat = jnp.concatenate([inputs, target, negatives], axis=1)
    idxcat = idxcat.astype(jnp.int32)
    idx3 = _transpose_idx(idxcat)
    table_flat = _repack_table(emb_table.T).reshape(TROWS, EMB)
    gathered = _make_sc_gather()(table_flat, idx3)
    g4 = gathered.reshape(ENT, B4, GSZ)

    eye4 = jnp.eye(4, dtype=jnp.float32)
    W1bd = jnp.kron(eye4, W1)
    b1t = jnp.tile(b1, 4).reshape(1, 128)
    W2p = jnp.stack(
        [jnp.kron(eye4, W2[i * EMB:(i + 1) * EMB, :]) for i in range(NE)]
    )
    b2t = jnp.tile(b2, 4).reshape(1, 4 * NE)

    out84 = _tc_dense(g4, W1bd, b1t, W2p, b2t)
    return out84.reshape(B, NE)


# speedup vs baseline: 2.3740x; 1.0284x over previous
import functools

import jax
import jax.numpy as jnp
from jax import lax
from jax.experimental import pallas as pl
from jax.experimental.pallas import tpu as pltpu
from jax.experimental.pallas import tpu_sc as plsc

EMB = 32
WIN = 5
NEG = 20
B = 16384
ENT = 2 * WIN + 1 + NEG
NW = 32
GSZ = 128
NCH = ENT * (B // GSZ)
NG = NCH // NW
NB = 8
B4 = B // 4
NE = NEG + 1


def _tr_body(i_ref, o_ref):
    v = i_ref[...]
    g = (v & -8192) + ((v & 2047) << 2) + ((v & 8191) >> 11)
    o_ref[...] = g.T.reshape(ENT, 8, GSZ)


def _transpose_idx(idxcat):
    return pl.pallas_call(
        _tr_body,
        grid=(B // 1024,),
        in_specs=[pl.BlockSpec((1024, ENT), lambda j: (j, 0))],
        out_specs=pl.BlockSpec((ENT, 8, GSZ), lambda j: (0, j, 0)),
        out_shape=jax.ShapeDtypeStruct((ENT, B // GSZ, GSZ), jnp.int32),
    )(idxcat)


VOCAB = 1000000
RB = 8192
NRB = (VOCAB + RB - 1) // RB
TROWS = NRB * RB


def _rp_body(t_ref, o_ref):
    q = RB // 4
    for a in range(4):
        o_ref[:, a * EMB:(a + 1) * EMB] = t_ref[:, a * q:(a + 1) * q].T


def _repack_table(tT):
    return pl.pallas_call(
        _rp_body,
        grid=(NRB,),
        in_specs=[pl.BlockSpec((EMB, RB), lambda j: (0, j))],
        out_specs=pl.BlockSpec((RB // 4, 128), lambda j: (j, 0)),
        out_shape=jax.ShapeDtypeStruct((TROWS // 4, 128), jnp.float32),
    )(tT)


@functools.lru_cache(maxsize=2)
def _make_sc_gather(bq0, nbq):
    mesh = plsc.VectorSubcoreMesh(core_axis_name="c", subcore_axis_name="s")
    ng = ENT * nbq // NW

    @functools.partial(
        pl.kernel,
        mesh=mesh,
        out_type=jax.ShapeDtypeStruct((ENT, nbq * GSZ, EMB), jnp.float32),
        scratch_types=[
            pltpu.VMEM((NB, GSZ), jnp.int32),
            pltpu.VMEM((NB, GSZ, EMB), jnp.float32),
            pltpu.SemaphoreType.DMA((NB,)),
            pltpu.SemaphoreType.DMA((NB,)),
        ],
        compiler_params=pltpu.CompilerParams(use_tc_tiling_on_sc=False),
    )
    def _sc_gather(table, idx3, out, idx_v, rows_v, isem, gsem):
        wid = lax.axis_index("s") * 2 + lax.axis_index("c")
        c0 = wid * ng
        NG = ng

        def idx_src(c):
            return idx3.at[c // nbq, bq0 + lax.rem(c, nbq)]

        def fire_idx(c, b):
            pltpu.async_copy(idx_src(c), idx_v.at[b], isem.at[b])

        def wait_idx(c, b):
            pltpu.make_async_copy(idx_src(c), idx_v.at[b], isem.at[b]).wait()

        def fire_gather(b):
            pltpu.async_copy(table.at[idx_v.at[b]], rows_v.at[b], gsem.at[b])

        def wait_gather(b):
            pltpu.make_async_copy(
                table.at[idx_v.at[b]], rows_v.at[b], gsem.at[b]
            ).wait()

        def store(c, b):
            pltpu.sync_copy(
                rows_v.at[b],
                out.at[c // nbq, pl.ds(lax.rem(c, nbq) * GSZ, GSZ)],
            )

        for b in range(NB):
            fire_idx(c0 + b, b)
        for b in range(NB):
            wait_idx(c0 + b, b)
            fire_gather(b)

        def outer(o, carry):
            for b in range(NB):
                r = o * NB + b
                c = c0 + r
                wait_gather(b)
                store(c, b)

                @pl.when(r + NB < NG)
                def _refill():
                    fire_idx(c + NB, b)
                    wait_idx(c + NB, b)
                    fire_gather(b)

            return carry

        lax.fori_loop(0, NG // NB, outer, 0)

        rem = NG - (NG // NB) * NB
        for b in range(rem):
            c = c0 + (NG // NB) * NB + b
            wait_gather(b)
            store(c, b)

    return _sc_gather


BQ = 256


def _tc_body(g_ref, w1_ref, b1_ref, w2_ref, b2_ref, o_ref):
    g = g_ref[...].reshape(ENT * BQ, 128)
    h = jax.nn.sigmoid(
        jnp.dot(g, w1_ref[...], preferred_element_type=jnp.float32)
        + b1_ref[...]
    )
    h = h.reshape(ENT, BQ, 128)
    means = h[0]
    for i in range(1, 2 * WIN):
        means = means + h[i]
    means = means * (1.0 / (2 * WIN))
    acc = jnp.dot(means * h[2 * WIN], w2_ref[0],
                  preferred_element_type=jnp.float32)
    for i in range(1, NE):
        acc = acc + jnp.dot(means * h[2 * WIN + i], w2_ref[i],
                            preferred_element_type=jnp.float32)
    logits = acc + b2_ref[...]
    for k in range(4):
        seg = logits[:, NE * k:NE * (k + 1)]
        m = jnp.max(seg, axis=-1, keepdims=True)
        ex = jnp.exp(seg - m)
        o_ref[:, NE * k:NE * (k + 1)] = ex / jnp.sum(ex, axis=-1,
                                                     keepdims=True)


def _tc_dense(g4, W1bd, b1t, W2p, b2t):
    n4 = g4.shape[1]
    return pl.pallas_call(
        _tc_body,
        grid=(n4 // BQ,),
        in_specs=[
            pl.BlockSpec((ENT, BQ, 128), lambda i: (0, i, 0)),
            pl.BlockSpec((128, 128), lambda i: (0, 0)),
            pl.BlockSpec((1, 128), lambda i: (0, 0)),
            pl.BlockSpec((NE, 128, 4 * NE), lambda i: (0, 0, 0)),
            pl.BlockSpec((1, 4 * NE), lambda i: (0, 0)),
        ],
        out_specs=pl.BlockSpec((BQ, 4 * NE), lambda i: (i, 0)),
        out_shape=jax.ShapeDtypeStruct((n4, 4 * NE), jnp.float32),
    )(g4, W1bd, b1t, W2p, b2t)


def kernel(inputs, target, negatives, emb_table, W1, b1, W2, b2):
    idxcat = jnp.concatenate([inputs, target, negatives], axis=1)
    idxcat = idxcat.astype(jnp.int32)
    idx3 = _transpose_idx(idxcat)
    table_flat = _repack_table(emb_table.T).reshape(TROWS, EMB)

    eye4 = jnp.eye(4, dtype=jnp.float32)
    W1bd = jnp.kron(eye4, W1)
    b1t = jnp.tile(b1, 4).reshape(1, 128)
    W2p = jnp.stack(
        [jnp.kron(eye4, W2[i * EMB:(i + 1) * EMB, :]) for i in range(NE)]
    )
    b2t = jnp.tile(b2, 4).reshape(1, 4 * NE)

    nbq = (B // GSZ) // 2
    outs = []
    for h in range(2):
        gathered = _make_sc_gather(h * nbq, nbq)(table_flat, idx3)
        g4 = gathered.reshape(ENT, nbq * GSZ // 4, GSZ)
        outs.append(_tc_dense(g4, W1bd, b1t, W2p, b2t))
    return jnp.concatenate(outs, axis=0).reshape(B, NE)
